# Initial kernel scaffold; baseline (speedup 1.0000x reference)
import jax
import jax.numpy as jnp
from jax.experimental import pallas as pl


def kernel(query_embeddings, candidate_embeddings, candidate_labels, W_q, W_k, W_v, W_o, W_l, b_l):
    raise NotImplementedError("write your pallas kernel here")


# speedup vs baseline: 7.3096x; 7.3096x over previous
import functools
import jax
import jax.numpy as jnp
from jax.experimental import pallas as pl


K_NEIGH = 96
INV_SCALE = 1.0 / 16.0
CH = 512


def _mm_kernel(a_ref, b_ref, o_ref, *, dn, precision):
    o_ref[...] = jax.lax.dot_general(
        a_ref[...], b_ref[...], dn, precision=precision,
        preferred_element_type=jnp.float32)


def _proj(a, w, transpose_w, bn=512, precision=None):
    M, K = a.shape
    N = w.shape[0] if transpose_w else w.shape[1]
    dn = ((((1,), (1,)) if transpose_w else ((1,), (0,))), ((), ()))
    wspec = (pl.BlockSpec((bn, K), lambda j: (j, 0)) if transpose_w
             else pl.BlockSpec((K, bn), lambda j: (0, j)))
    return pl.pallas_call(
        functools.partial(_mm_kernel, dn=dn, precision=precision),
        grid=(N // bn,),
        in_specs=[pl.BlockSpec((M, K), lambda j: (0, 0)), wspec],
        out_specs=pl.BlockSpec((M, bn), lambda j: (0, j)),
        out_shape=jax.ShapeDtypeStruct((M, N), jnp.float32),
    )(a, w)


def _proj_rows(a, w, bm=1024, bn=512):
    M, K = a.shape
    N = w.shape[0]
    return pl.pallas_call(
        functools.partial(_mm_kernel, dn=(((1,), (1,)), ((), ())),
                          precision=None),
        grid=(M // bm, N // bn),
        in_specs=[pl.BlockSpec((bm, K), lambda i, j: (i, 0)),
                  pl.BlockSpec((bn, K), lambda i, j: (j, 0))],
        out_specs=pl.BlockSpec((bm, bn), lambda i, j: (i, j)),
        out_shape=jax.ShapeDtypeStruct((M, N), jnp.float32),
    )(a, w)


def _scores_kernel(p_ref, k_ref, s_ref):
    s = jax.lax.dot_general(
        p_ref[...], k_ref[...], (((1,), (1,)), ((), ())),
        preferred_element_type=jnp.float32)
    s_ref[...] = s * INV_SCALE


def _scores(p, k2d):
    M, K = p.shape
    NCAND = k2d.shape[0]
    nch = NCAND // CH
    return pl.pallas_call(
        _scores_kernel,
        grid=(nch,),
        in_specs=[pl.BlockSpec((M, K), lambda j: (0, 0)),
                  pl.BlockSpec((CH, K), lambda j: (j, 0))],
        out_specs=pl.BlockSpec((M, CH), lambda j: (0, j)),
        out_shape=jax.ShapeDtypeStruct((M, NCAND), jnp.float32),
    )(p, k2d)


def _key(i):
    return i ^ ((i >> 31) & jnp.int32(0x7FFFFFFF))


def _f2key(f):
    return _key(jax.lax.bitcast_convert_type(f, jnp.int32))


def _key2f(k):
    return jax.lax.bitcast_convert_type(_key(k), jnp.float32)


def _thresh_kernel(s_ref, t_ref, m_ref, *, nch, rb):
    rmax = jnp.full((rb, 1), -3.0e38, jnp.float32)
    rmin = jnp.full((rb, 1), 3.0e38, jnp.float32)
    for c in range(nch):
        x = s_ref[:, c * CH:(c + 1) * CH]
        rmax = jnp.maximum(rmax, jnp.max(x, axis=1, keepdims=True))
        rmin = jnp.minimum(rmin, jnp.min(x, axis=1, keepdims=True))

    lo = _f2key(rmin)
    hi = _f2key(rmax)

    def bis_body(_, carry):
        lo, hi = carry
        floor_avg = (lo >> 1) + (hi >> 1) + (lo & hi & 1)
        mid = floor_avg + ((lo ^ hi) & 1)
        midf = _key2f(mid)
        cnt = jnp.zeros((rb, 1), jnp.int32)
        for c in range(nch):
            x = s_ref[:, c * CH:(c + 1) * CH]
            cnt = cnt + jnp.sum((x >= midf).astype(jnp.int32),
                                axis=1, keepdims=True)
        pred = cnt >= K_NEIGH
        return (jnp.where(pred, mid, lo), jnp.where(pred, hi, mid - 1))

    lo, hi = jax.lax.fori_loop(0, 32, bis_body, (lo, hi))
    t_ref[...] = _key2f(lo)
    m_ref[...] = rmax


def _thresholds(s2d, rb=128):
    M, NCAND = s2d.shape
    nch = NCAND // CH
    out_t = jax.ShapeDtypeStruct((M, 1), jnp.float32)
    return pl.pallas_call(
        functools.partial(_thresh_kernel, nch=nch, rb=rb),
        grid=(M // rb,),
        in_specs=[pl.BlockSpec((rb, NCAND), lambda i: (i, 0))],
        out_specs=[pl.BlockSpec((rb, 1), lambda i: (i, 0)),
                   pl.BlockSpec((rb, 1), lambda i: (i, 0))],
        out_shape=[out_t, out_t],
    )(s2d)


def _ctx_kernel(s_ref, t_ref, m_ref, k_ref, l_ref, cr_ref, cl_ref, dn_ref):
    j = pl.program_id(0)
    x = s_ref[...]
    w = jnp.where(x >= t_ref[...], jnp.exp(x - m_ref[...]), 0.0)
    part = jax.lax.dot_general(w, k_ref[...], (((1,), (0,)), ((), ())),
                               preferred_element_type=jnp.float32)
    lpart = jax.lax.dot_general(w, l_ref[...], (((1,), (0,)), ((), ())),
                                preferred_element_type=jnp.float32)
    dpart = jnp.sum(w, axis=1, keepdims=True)

    @pl.when(j == 0)
    def _():
        cr_ref[...] = part
        cl_ref[...] = lpart
        dn_ref[...] = dpart

    @pl.when(j > 0)
    def _():
        cr_ref[...] += part
        cl_ref[...] += lpart
        dn_ref[...] += dpart


def _context(s2d, t, m, k2d, labels2d):
    M = s2d.shape[0]
    NCAND, D = k2d.shape
    nch = NCAND // CH
    col = jax.ShapeDtypeStruct((M, 1), jnp.float32)
    return pl.pallas_call(
        _ctx_kernel,
        grid=(nch,),
        in_specs=[pl.BlockSpec((M, CH), lambda j: (0, j)),
                  pl.BlockSpec((M, 1), lambda j: (0, 0)),
                  pl.BlockSpec((M, 1), lambda j: (0, 0)),
                  pl.BlockSpec((CH, D), lambda j: (j, 0)),
                  pl.BlockSpec((CH, 1), lambda j: (j, 0))],
        out_specs=[pl.BlockSpec((M, D), lambda j: (0, 0)),
                   pl.BlockSpec((M, 1), lambda j: (0, 0)),
                   pl.BlockSpec((M, 1), lambda j: (0, 0))],
        out_shape=[jax.ShapeDtypeStruct((M, D), jnp.float32), col, col],
    )(s2d, t, m, k2d, labels2d)


def _out1_kernel(cr_ref, wv_ref, wvl_ref, cl_ref, dn_ref, o_ref):
    ctx = jax.lax.dot_general(cr_ref[...], wv_ref[...], (((1,), (1,)), ((), ())),
                              preferred_element_type=jnp.float32)
    ctx = ctx + jax.lax.dot_general(cl_ref[...], wvl_ref[...],
                                    (((1,), (1,)), ((), ())),
                                    preferred_element_type=jnp.float32)
    o_ref[...] = ctx / dn_ref[...]


def _out1(cr, wv48, wvl, cl, dn, bn=512):
    M, D = cr.shape
    return pl.pallas_call(
        _out1_kernel,
        grid=(D // bn,),
        in_specs=[pl.BlockSpec((M, D), lambda j: (0, 0)),
                  pl.BlockSpec((bn, D), lambda j: (j, 0)),
                  pl.BlockSpec((bn, 1), lambda j: (j, 0)),
                  pl.BlockSpec((M, 1), lambda j: (0, 0)),
                  pl.BlockSpec((M, 1), lambda j: (0, 0))],
        out_specs=pl.BlockSpec((M, bn), lambda j: (0, j)),
        out_shape=jax.ShapeDtypeStruct((M, D), jnp.float32),
    )(cr, wv48, wvl, cl, dn)


def kernel(query_embeddings, candidate_embeddings, candidate_labels,
           W_q, W_k, W_v, W_o, W_l, b_l):
    B = query_embeddings.shape[0]
    NCAND = candidate_embeddings.shape[0]
    D = W_q.shape[0]
    q2d = query_embeddings.reshape(B, D)
    k2d = candidate_embeddings.reshape(NCAND, D)
    labels2d = candidate_labels.reshape(NCAND, 1)

    qp = _proj(q2d, W_q, transpose_w=True)
    kp = _proj_rows(k2d, W_k)
    s2d = _scores(qp, kp)
    t, m = _thresholds(s2d)
    cr, cl, dn = _context(s2d, t, m, k2d, labels2d)
    ctx = _out1(cr, W_v[:, :D], W_v[:, D:], cl, dn)
    return _proj(ctx, W_o, transpose_w=True)


# speedup vs baseline: 7.5031x; 1.0265x over previous
import functools
import jax
import jax.numpy as jnp
from jax.experimental import pallas as pl


K_NEIGH = 96
INV_SCALE = 1.0 / 16.0
CH = 512


def _mm_kernel(a_ref, b_ref, o_ref, *, dn, precision):
    o_ref[...] = jax.lax.dot_general(
        a_ref[...], b_ref[...], dn, precision=precision,
        preferred_element_type=jnp.float32)


def _proj(a, w, transpose_w, bn=512, precision=None):
    M, K = a.shape
    N = w.shape[0] if transpose_w else w.shape[1]
    dn = ((((1,), (1,)) if transpose_w else ((1,), (0,))), ((), ()))
    wspec = (pl.BlockSpec((bn, K), lambda j: (j, 0)) if transpose_w
             else pl.BlockSpec((K, bn), lambda j: (0, j)))
    return pl.pallas_call(
        functools.partial(_mm_kernel, dn=dn, precision=precision),
        grid=(N // bn,),
        in_specs=[pl.BlockSpec((M, K), lambda j: (0, 0)), wspec],
        out_specs=pl.BlockSpec((M, bn), lambda j: (0, j)),
        out_shape=jax.ShapeDtypeStruct((M, N), jnp.float32),
    )(a, w)


def _proj_rows(a, w, bm=1024, bn=512):
    M, K = a.shape
    N = w.shape[0]
    return pl.pallas_call(
        functools.partial(_mm_kernel, dn=(((1,), (1,)), ((), ())),
                          precision=None),
        grid=(M // bm, N // bn),
        in_specs=[pl.BlockSpec((bm, K), lambda i, j: (i, 0)),
                  pl.BlockSpec((bn, K), lambda i, j: (j, 0))],
        out_specs=pl.BlockSpec((bm, bn), lambda i, j: (i, j)),
        out_shape=jax.ShapeDtypeStruct((M, N), jnp.float32),
    )(a, w)


def _scores_kernel(p_ref, k_ref, s_ref):
    s = jax.lax.dot_general(
        p_ref[...], k_ref[...], (((1,), (1,)), ((), ())),
        preferred_element_type=jnp.float32)
    s_ref[...] = s * INV_SCALE


def _scores(p, k2d):
    M, K = p.shape
    NCAND = k2d.shape[0]
    nch = NCAND // CH
    return pl.pallas_call(
        _scores_kernel,
        grid=(nch,),
        in_specs=[pl.BlockSpec((M, K), lambda j: (0, 0)),
                  pl.BlockSpec((CH, K), lambda j: (j, 0))],
        out_specs=pl.BlockSpec((M, CH), lambda j: (0, j)),
        out_shape=jax.ShapeDtypeStruct((M, NCAND), jnp.float32),
    )(p, k2d)


def _key(i):
    return i ^ ((i >> 31) & jnp.int32(0x7FFFFFFF))


def _f2key(f):
    return _key(jax.lax.bitcast_convert_type(f, jnp.int32))


def _key2f(k):
    return jax.lax.bitcast_convert_type(_key(k), jnp.float32)


def _thresh_kernel(s_ref, ones_ref, t_ref, m_ref, *, nch, rb):
    rmax = jnp.full((rb, 1), -3.0e38, jnp.float32)
    rmin = jnp.full((rb, 1), 3.0e38, jnp.float32)
    for c in range(nch):
        x = s_ref[:, c * CH:(c + 1) * CH]
        rmax = jnp.maximum(rmax, jnp.max(x, axis=1, keepdims=True))
        rmin = jnp.minimum(rmin, jnp.min(x, axis=1, keepdims=True))

    lo = _f2key(rmin)
    hi = _f2key(rmax)

    def bis_body(_, carry):
        lo, hi = carry
        floor_avg = (lo >> 1) + (hi >> 1) + (lo & hi & 1)
        mid = floor_avg + ((lo ^ hi) & 1)
        midf = _key2f(mid)
        ind = jnp.zeros((rb, 1), jnp.float32)
        for c in range(nch):
            x = s_ref[:, c * CH:(c + 1) * CH]
            i = jnp.where(x >= midf, 1.0, 0.0)
            ind = ind + jax.lax.dot_general(
                i, ones_ref[c * CH:(c + 1) * CH, :],
                (((1,), (0,)), ((), ())),
                preferred_element_type=jnp.float32)
        pred = ind >= float(K_NEIGH)
        return (jnp.where(pred, mid, lo), jnp.where(pred, hi, mid - 1))

    lo, hi = jax.lax.fori_loop(0, 32, bis_body, (lo, hi))
    t_ref[...] = _key2f(lo)
    m_ref[...] = rmax


def _thresholds(s2d, rb=128):
    M, NCAND = s2d.shape
    nch = NCAND // CH
    out_t = jax.ShapeDtypeStruct((M, 1), jnp.float32)
    onescol = jnp.ones((NCAND, 1), jnp.float32)
    return pl.pallas_call(
        functools.partial(_thresh_kernel, nch=nch, rb=rb),
        grid=(M // rb,),
        in_specs=[pl.BlockSpec((rb, NCAND), lambda i: (i, 0)),
                  pl.BlockSpec((NCAND, 1), lambda i: (0, 0))],
        out_specs=[pl.BlockSpec((rb, 1), lambda i: (i, 0)),
                   pl.BlockSpec((rb, 1), lambda i: (i, 0))],
        out_shape=[out_t, out_t],
    )(s2d, onescol)


def _ctx_kernel(s_ref, t_ref, m_ref, k_ref, l_ref, cr_ref, cl_ref, dn_ref):
    j = pl.program_id(0)
    x = s_ref[...]
    w = jnp.where(x >= t_ref[...], jnp.exp(x - m_ref[...]), 0.0)
    part = jax.lax.dot_general(w, k_ref[...], (((1,), (0,)), ((), ())),
                               preferred_element_type=jnp.float32)
    lpart = jax.lax.dot_general(w, l_ref[...], (((1,), (0,)), ((), ())),
                                preferred_element_type=jnp.float32)
    dpart = jnp.sum(w, axis=1, keepdims=True)

    @pl.when(j == 0)
    def _():
        cr_ref[...] = part
        cl_ref[...] = lpart
        dn_ref[...] = dpart

    @pl.when(j > 0)
    def _():
        cr_ref[...] += part
        cl_ref[...] += lpart
        dn_ref[...] += dpart


def _context(s2d, t, m, k2d, labels2d):
    M = s2d.shape[0]
    NCAND, D = k2d.shape
    nch = NCAND // CH
    col = jax.ShapeDtypeStruct((M, 1), jnp.float32)
    return pl.pallas_call(
        _ctx_kernel,
        grid=(nch,),
        in_specs=[pl.BlockSpec((M, CH), lambda j: (0, j)),
                  pl.BlockSpec((M, 1), lambda j: (0, 0)),
                  pl.BlockSpec((M, 1), lambda j: (0, 0)),
                  pl.BlockSpec((CH, D), lambda j: (j, 0)),
                  pl.BlockSpec((CH, 1), lambda j: (j, 0))],
        out_specs=[pl.BlockSpec((M, D), lambda j: (0, 0)),
                   pl.BlockSpec((M, 1), lambda j: (0, 0)),
                   pl.BlockSpec((M, 1), lambda j: (0, 0))],
        out_shape=[jax.ShapeDtypeStruct((M, D), jnp.float32), col, col],
    )(s2d, t, m, k2d, labels2d)


def _out1_kernel(cr_ref, wv_ref, wvl_ref, cl_ref, dn_ref, o_ref):
    ctx = jax.lax.dot_general(cr_ref[...], wv_ref[...], (((1,), (1,)), ((), ())),
                              preferred_element_type=jnp.float32)
    ctx = ctx + jax.lax.dot_general(cl_ref[...], wvl_ref[...],
                                    (((1,), (1,)), ((), ())),
                                    preferred_element_type=jnp.float32)
    o_ref[...] = ctx / dn_ref[...]


def _out1(cr, wv48, wvl, cl, dn, bn=512):
    M, D = cr.shape
    return pl.pallas_call(
        _out1_kernel,
        grid=(D // bn,),
        in_specs=[pl.BlockSpec((M, D), lambda j: (0, 0)),
                  pl.BlockSpec((bn, D), lambda j: (j, 0)),
                  pl.BlockSpec((bn, 1), lambda j: (j, 0)),
                  pl.BlockSpec((M, 1), lambda j: (0, 0)),
                  pl.BlockSpec((M, 1), lambda j: (0, 0))],
        out_specs=pl.BlockSpec((M, bn), lambda j: (0, j)),
        out_shape=jax.ShapeDtypeStruct((M, D), jnp.float32),
    )(cr, wv48, wvl, cl, dn)


def kernel(query_embeddings, candidate_embeddings, candidate_labels,
           W_q, W_k, W_v, W_o, W_l, b_l):
    B = query_embeddings.shape[0]
    NCAND = candidate_embeddings.shape[0]
    D = W_q.shape[0]
    q2d = query_embeddings.reshape(B, D)
    k2d = candidate_embeddings.reshape(NCAND, D)
    labels2d = candidate_labels.reshape(NCAND, 1)

    qp = _proj(q2d, W_q, transpose_w=True)
    kp = _proj_rows(k2d, W_k)
    s2d = _scores(qp, kp)
    t, m = _thresholds(s2d)
    cr, cl, dn = _context(s2d, t, m, k2d, labels2d)
    ctx = _out1(cr, W_v[:, :D], W_v[:, D:], cl, dn)
    return _proj(ctx, W_o, transpose_w=True)


# speedup vs baseline: 7.8988x; 1.0527x over previous
import functools
import jax
import jax.numpy as jnp
from jax.experimental import pallas as pl


K_NEIGH = 96
INV_SCALE = 1.0 / 16.0
CH = 512


def _mm_kernel(a_ref, b_ref, o_ref, *, dn, precision):
    o_ref[...] = jax.lax.dot_general(
        a_ref[...], b_ref[...], dn, precision=precision,
        preferred_element_type=jnp.float32)


def _proj(a, w, transpose_w, bn=512, precision=None):
    M, K = a.shape
    N = w.shape[0] if transpose_w else w.shape[1]
    dn = ((((1,), (1,)) if transpose_w else ((1,), (0,))), ((), ()))
    wspec = (pl.BlockSpec((bn, K), lambda j: (j, 0)) if transpose_w
             else pl.BlockSpec((K, bn), lambda j: (0, j)))
    return pl.pallas_call(
        functools.partial(_mm_kernel, dn=dn, precision=precision),
        grid=(N // bn,),
        in_specs=[pl.BlockSpec((M, K), lambda j: (0, 0)), wspec],
        out_specs=pl.BlockSpec((M, bn), lambda j: (0, j)),
        out_shape=jax.ShapeDtypeStruct((M, N), jnp.float32),
    )(a, w)


def _proj_rows(a, w, bm=1024, bn=512):
    M, K = a.shape
    N = w.shape[0]
    return pl.pallas_call(
        functools.partial(_mm_kernel, dn=(((1,), (1,)), ((), ())),
                          precision=None),
        grid=(M // bm, N // bn),
        in_specs=[pl.BlockSpec((bm, K), lambda i, j: (i, 0)),
                  pl.BlockSpec((bn, K), lambda i, j: (j, 0))],
        out_specs=pl.BlockSpec((bm, bn), lambda i, j: (i, j)),
        out_shape=jax.ShapeDtypeStruct((M, N), jnp.float32),
    )(a, w)


def _scores_kernel(p_ref, k_ref, s_ref):
    s = jax.lax.dot_general(
        p_ref[...], k_ref[...], (((1,), (1,)), ((), ())),
        preferred_element_type=jnp.float32)
    s_ref[...] = s * INV_SCALE


def _scores(p, k2d):
    M, K = p.shape
    NCAND = k2d.shape[0]
    nch = NCAND // CH
    return pl.pallas_call(
        _scores_kernel,
        grid=(nch,),
        in_specs=[pl.BlockSpec((M, K), lambda j: (0, 0)),
                  pl.BlockSpec((CH, K), lambda j: (j, 0))],
        out_specs=pl.BlockSpec((M, CH), lambda j: (0, j)),
        out_shape=jax.ShapeDtypeStruct((M, NCAND), jnp.float32),
    )(p, k2d)


def _key(i):
    return i ^ ((i >> 31) & jnp.int32(0x7FFFFFFF))


def _f2key(f):
    return _key(jax.lax.bitcast_convert_type(f, jnp.int32))


def _key2f(k):
    return jax.lax.bitcast_convert_type(_key(k), jnp.float32)


def _thresh_kernel(s_ref, ones_ref, t_ref, m_ref, *, nch, rb):
    rmax = jnp.full((rb, 1), -3.0e38, jnp.float32)
    rmin = jnp.full((rb, 1), 3.0e38, jnp.float32)
    for c in range(nch):
        x = s_ref[:, c * CH:(c + 1) * CH]
        rmax = jnp.maximum(rmax, jnp.max(x, axis=1, keepdims=True))
        rmin = jnp.minimum(rmin, jnp.min(x, axis=1, keepdims=True))

    lo = _f2key(rmin)
    hi = _f2key(rmax)

    def bis_cond(carry):
        lo, hi = carry
        return jnp.any(lo < hi)

    def bis_body(carry):
        lo, hi = carry
        floor_avg = (lo >> 1) + (hi >> 1) + (lo & hi & 1)
        mid = floor_avg + ((lo ^ hi) & 1)
        midf = _key2f(mid)
        ind = jnp.zeros((rb, 1), jnp.float32)
        for c in range(nch):
            x = s_ref[:, c * CH:(c + 1) * CH]
            i = jnp.where(x >= midf, 1.0, 0.0)
            ind = ind + jax.lax.dot_general(
                i, ones_ref[c * CH:(c + 1) * CH, :],
                (((1,), (0,)), ((), ())),
                preferred_element_type=jnp.float32)
        pred = ind >= float(K_NEIGH)
        lo = jnp.where(pred, mid, lo)
        hi = jnp.where(pred, hi, mid - 1)
        eq = ind == float(K_NEIGH)
        hi = jnp.where(eq, mid, hi)
        return lo, hi

    lo, hi = jax.lax.while_loop(bis_cond, bis_body, (lo, hi))
    t_ref[...] = _key2f(lo)
    m_ref[...] = rmax


def _thresholds(s2d, rb=128):
    M, NCAND = s2d.shape
    nch = NCAND // CH
    out_t = jax.ShapeDtypeStruct((M, 1), jnp.float32)
    onescol = jnp.ones((NCAND, 1), jnp.float32)
    return pl.pallas_call(
        functools.partial(_thresh_kernel, nch=nch, rb=rb),
        grid=(M // rb,),
        in_specs=[pl.BlockSpec((rb, NCAND), lambda i: (i, 0)),
                  pl.BlockSpec((NCAND, 1), lambda i: (0, 0))],
        out_specs=[pl.BlockSpec((rb, 1), lambda i: (i, 0)),
                   pl.BlockSpec((rb, 1), lambda i: (i, 0))],
        out_shape=[out_t, out_t],
    )(s2d, onescol)


def _ctx_kernel(s_ref, t_ref, m_ref, k_ref, l_ref, cr_ref, cl_ref, dn_ref):
    j = pl.program_id(0)
    x = s_ref[...]
    w = jnp.where(x >= t_ref[...], jnp.exp(x - m_ref[...]), 0.0)
    part = jax.lax.dot_general(w, k_ref[...], (((1,), (0,)), ((), ())),
                               preferred_element_type=jnp.float32)
    lpart = jax.lax.dot_general(w, l_ref[...], (((1,), (0,)), ((), ())),
                                preferred_element_type=jnp.float32)
    dpart = jnp.sum(w, axis=1, keepdims=True)

    @pl.when(j == 0)
    def _():
        cr_ref[...] = part
        cl_ref[...] = lpart
        dn_ref[...] = dpart

    @pl.when(j > 0)
    def _():
        cr_ref[...] += part
        cl_ref[...] += lpart
        dn_ref[...] += dpart


def _context(s2d, t, m, k2d, labels2d):
    M = s2d.shape[0]
    NCAND, D = k2d.shape
    nch = NCAND // CH
    col = jax.ShapeDtypeStruct((M, 1), jnp.float32)
    return pl.pallas_call(
        _ctx_kernel,
        grid=(nch,),
        in_specs=[pl.BlockSpec((M, CH), lambda j: (0, j)),
                  pl.BlockSpec((M, 1), lambda j: (0, 0)),
                  pl.BlockSpec((M, 1), lambda j: (0, 0)),
                  pl.BlockSpec((CH, D), lambda j: (j, 0)),
                  pl.BlockSpec((CH, 1), lambda j: (j, 0))],
        out_specs=[pl.BlockSpec((M, D), lambda j: (0, 0)),
                   pl.BlockSpec((M, 1), lambda j: (0, 0)),
                   pl.BlockSpec((M, 1), lambda j: (0, 0))],
        out_shape=[jax.ShapeDtypeStruct((M, D), jnp.float32), col, col],
    )(s2d, t, m, k2d, labels2d)


def _out1_kernel(cr_ref, wv_ref, wvl_ref, cl_ref, dn_ref, o_ref):
    ctx = jax.lax.dot_general(cr_ref[...], wv_ref[...], (((1,), (1,)), ((), ())),
                              preferred_element_type=jnp.float32)
    ctx = ctx + jax.lax.dot_general(cl_ref[...], wvl_ref[...],
                                    (((1,), (1,)), ((), ())),
                                    preferred_element_type=jnp.float32)
    o_ref[...] = ctx / dn_ref[...]


def _out1(cr, wv48, wvl, cl, dn, bn=512):
    M, D = cr.shape
    return pl.pallas_call(
        _out1_kernel,
        grid=(D // bn,),
        in_specs=[pl.BlockSpec((M, D), lambda j: (0, 0)),
                  pl.BlockSpec((bn, D), lambda j: (j, 0)),
                  pl.BlockSpec((bn, 1), lambda j: (j, 0)),
                  pl.BlockSpec((M, 1), lambda j: (0, 0)),
                  pl.BlockSpec((M, 1), lambda j: (0, 0))],
        out_specs=pl.BlockSpec((M, bn), lambda j: (0, j)),
        out_shape=jax.ShapeDtypeStruct((M, D), jnp.float32),
    )(cr, wv48, wvl, cl, dn)


def kernel(query_embeddings, candidate_embeddings, candidate_labels,
           W_q, W_k, W_v, W_o, W_l, b_l):
    B = query_embeddings.shape[0]
    NCAND = candidate_embeddings.shape[0]
    D = W_q.shape[0]
    q2d = query_embeddings.reshape(B, D)
    k2d = candidate_embeddings.reshape(NCAND, D)
    labels2d = candidate_labels.reshape(NCAND, 1)

    qp = _proj(q2d, W_q, transpose_w=True)
    kp = _proj_rows(k2d, W_k)
    s2d = _scores(qp, kp)
    t, m = _thresholds(s2d)
    cr, cl, dn = _context(s2d, t, m, k2d, labels2d)
    ctx = _out1(cr, W_v[:, :D], W_v[:, D:], cl, dn)
    return _proj(ctx, W_o, transpose_w=True)
